# baseline (speedup 1.0000x reference)
import functools

import jax
import jax.numpy as jnp
from jax import lax
from jax.experimental import pallas as pl
from jax.experimental.pallas import tpu as pltpu
from jax.experimental.pallas import tpu_sc as plsc

_VOCAB = 50257
_HIDDEN = 1024
_B = 4
_S = 2048
_N = _B * _S
_NC = 2
_NS = 16
_NW = _NC * _NS
_P = _S // _NW
_C = 16
_JC = _P // _C
_NCH = _B * _JC
_LANES = 16


_NBUF = 3


def _emb_body(ids_hbm, word_hbm, pos_hbm, out_hbm, idx_v, rows_v, pos_v,
              gsem0, gsem1, gsem2, osem0, osem1, osem2, psem):
  gsems = (gsem0, gsem1, gsem2)
  osems = (osem0, osem1, osem2)
  wid = lax.axis_index("s") * _NC + lax.axis_index("c")
  pos_base = wid * _P

  pltpu.sync_copy(ids_hbm.at[wid], idx_v)
  pload = pltpu.async_copy(pos_hbm.at[pl.ds(pos_base, _P)], pos_v, psem)

  def gather(ch):
    return pltpu.async_copy(word_hbm.at[idx_v.at[ch]],
                            rows_v.at[ch % _NBUF], gsems[ch % _NBUF])

  gathers = [None] * _NBUF
  stores = [None] * _NBUF
  gathers[0] = gather(0)
  gathers[1] = gather(1)
  pload.wait()

  for ch in range(_NCH):
    cur = ch % _NBUF
    if ch + 2 < _NCH:
      pf = (ch + 2) % _NBUF
      if stores[pf] is not None:
        stores[pf].wait()
      gathers[pf] = gather(ch + 2)
    gathers[cur].wait()

    b, j = divmod(ch, _JC)

    def row_body(r, c2):
      for grp in range(_HIDDEN // _LANES):
        sl = pl.ds(grp * _LANES, _LANES)
        rows_v[cur, r, sl] += pos_v[j * _C + r, sl]
      return c2

    lax.fori_loop(0, _C, row_body, 0)

    out_off = b * _S + pos_base + j * _C
    stores[cur] = pltpu.async_copy(rows_v.at[cur],
                                   out_hbm.at[pl.ds(out_off, _C)],
                                   osems[cur])

  for st in stores:
    if st is not None:
      st.wait()


_mesh = plsc.VectorSubcoreMesh(
    core_axis_name="c", subcore_axis_name="s", num_cores=_NC,
    num_subcores=_NS)

_emb_kernel = functools.partial(
    pl.kernel,
    out_type=jax.ShapeDtypeStruct((_N, _HIDDEN), jnp.float32),
    mesh=_mesh,
    scratch_types=[
        pltpu.VMEM((_NCH, _C), jnp.int32),
        pltpu.VMEM((_NBUF, _C, _HIDDEN), jnp.float32),
        pltpu.VMEM((_P, _HIDDEN), jnp.float32),
        pltpu.SemaphoreType.DMA,
        pltpu.SemaphoreType.DMA,
        pltpu.SemaphoreType.DMA,
        pltpu.SemaphoreType.DMA,
        pltpu.SemaphoreType.DMA,
        pltpu.SemaphoreType.DMA,
        pltpu.SemaphoreType.DMA,
    ],
)(_emb_body)


@jax.jit
def kernel(input_ids, word_embeddings, position_embeddings):
  ids2 = (input_ids.reshape(_B, _NW, _JC, _C)
          .transpose(1, 0, 2, 3)
          .reshape(_NW, _NCH, _C))
  out = _emb_kernel(ids2, word_embeddings, position_embeddings)
  return out.reshape(_B, _S, _HIDDEN)


# speedup vs baseline: 1.4888x; 1.4888x over previous
import functools

import jax
import jax.numpy as jnp
from jax import lax
from jax.experimental import pallas as pl
from jax.experimental.pallas import tpu as pltpu
from jax.experimental.pallas import tpu_sc as plsc

_VOCAB = 50257
_HIDDEN = 1024
_B = 4
_S = 2048
_N = _B * _S
_NC = 2
_NS = 16
_NW = _NC * _NS
_P = _S // _NW
_PJ = 8
_CR = _B * _PJ
_NCH = _P // _PJ
_LANES = 16
_NBUF = 2


def _emb_body(ids_hbm, word_hbm, pos_hbm, out_hbm, idx_v, rows_v, pos_v,
              gsem0, gsem1, osem0, osem1, psem0, psem1):
  gsems = (gsem0, gsem1)
  osems = (osem0, osem1)
  psems = (psem0, psem1)
  wid = lax.axis_index("s") * _NC + lax.axis_index("c")
  pos_base = wid * _P

  pltpu.sync_copy(ids_hbm.at[wid], idx_v)

  def gather(ch):
    return pltpu.async_copy(word_hbm.at[idx_v.at[ch]],
                            rows_v.at[ch % _NBUF], gsems[ch % _NBUF])

  def pload(ch):
    return pltpu.async_copy(
        pos_hbm.at[pl.ds(pos_base + ch * _PJ, _PJ)],
        pos_v.at[ch % 2], psems[ch % 2])

  gathers = [None] * _NBUF
  ploads = [None, None]
  stores = [[None] * _B for _ in range(_NBUF)]

  gathers[0] = gather(0)
  ploads[0] = pload(0)

  for ch in range(_NCH):
    cur = ch % _NBUF
    nxt = (ch + 1) % _NBUF
    if ch + 1 < _NCH:
      for st in stores[nxt]:
        if st is not None:
          st.wait()
      gathers[nxt] = gather(ch + 1)
      ploads[(ch + 1) % 2] = pload(ch + 1)
    gathers[cur].wait()
    ploads[ch % 2].wait()

    def row_body(r, c2):
      for grp in range(_HIDDEN // _LANES):
        sl = pl.ds(grp * _LANES, _LANES)
        p = pos_v[ch % 2, r, sl]
        for b in range(_B):
          rows_v[cur, b * _PJ + r, sl] += p
      return c2

    lax.fori_loop(0, _PJ, row_body, 0)

    for b in range(_B):
      out_off = b * _S + pos_base + ch * _PJ
      stores[cur][b] = pltpu.async_copy(
          rows_v.at[cur, pl.ds(b * _PJ, _PJ)],
          out_hbm.at[pl.ds(out_off, _PJ)], osems[cur])

  for buf in stores:
    for st in buf:
      if st is not None:
        st.wait()


_mesh = plsc.VectorSubcoreMesh(
    core_axis_name="c", subcore_axis_name="s", num_cores=_NC,
    num_subcores=_NS)

_emb_kernel = functools.partial(
    pl.kernel,
    out_type=jax.ShapeDtypeStruct((_N, _HIDDEN), jnp.float32),
    mesh=_mesh,
    scratch_types=[
        pltpu.VMEM((_NCH, _CR), jnp.int32),
        pltpu.VMEM((_NBUF, _CR, _HIDDEN), jnp.float32),
        pltpu.VMEM((2, _PJ, _HIDDEN), jnp.float32),
        pltpu.SemaphoreType.DMA,
        pltpu.SemaphoreType.DMA,
        pltpu.SemaphoreType.DMA,
        pltpu.SemaphoreType.DMA,
        pltpu.SemaphoreType.DMA,
        pltpu.SemaphoreType.DMA,
    ],
)(_emb_body)


@jax.jit
def kernel(input_ids, word_embeddings, position_embeddings):
  ids3 = (input_ids.reshape(_B, _NW, _NCH, _PJ)
          .transpose(1, 2, 0, 3)
          .reshape(_NW, _NCH, _CR))
  out = _emb_kernel(ids3, word_embeddings, position_embeddings)
  return out.reshape(_B, _S, _HIDDEN)
